# Initial kernel scaffold; baseline (speedup 1.0000x reference)
import jax
import jax.numpy as jnp
from jax.experimental import pallas as pl


def kernel(token_ids, embedding_matrix):
    raise NotImplementedError("write your pallas kernel here")


# speedup vs baseline: 1.5842x; 1.5842x over previous
import functools

import jax
import jax.numpy as jnp
from jax import lax
from jax.experimental import pallas as pl
from jax.experimental.pallas import tpu as pltpu
from jax.experimental.pallas import tpu_sc as plsc

NC = 2
NS = 16
NW = NC * NS
CHUNK = 128


def _gather_kernel(n_rows, d):
    b_per_w = n_rows // NW
    n_chunks = b_per_w // CHUNK
    mesh = plsc.VectorSubcoreMesh(core_axis_name="c", subcore_axis_name="s")

    @functools.partial(
        pl.kernel,
        mesh=mesh,
        out_type=jax.ShapeDtypeStruct((n_rows, d), jnp.float32),
        scratch_types=[
            pltpu.VMEM((CHUNK,), jnp.int32),
            pltpu.VMEM((CHUNK, d), jnp.float32),
            pltpu.SemaphoreType.DMA,
        ],
        compiler_params=pltpu.CompilerParams(use_tc_tiling_on_sc=False),
    )
    def k(idx_hbm, table_hbm, out_hbm, idx_v, rows_v, sem):
        wid = lax.axis_index("s") * NC + lax.axis_index("c")
        base = wid * b_per_w

        def body(i, carry):
            off = base + i * CHUNK
            pltpu.sync_copy(idx_hbm.at[pl.ds(off, CHUNK)], idx_v)
            pltpu.async_copy(table_hbm.at[idx_v], rows_v, sem).wait()
            pltpu.sync_copy(rows_v, out_hbm.at[pl.ds(off, CHUNK)])
            return carry

        lax.fori_loop(0, n_chunks, body, 0)

    return k


def kernel(token_ids, embedding_matrix):
    b, s = token_ids.shape
    d = embedding_matrix.shape[1]
    idx = token_ids.reshape(-1).astype(jnp.int32)
    n = idx.shape[0]
    tile = NW * CHUNK
    n_pad = ((n + tile - 1) // tile) * tile
    if n_pad != n:
        idx = jnp.pad(idx, (0, n_pad - n))
    out = _gather_kernel(n_pad, d)(idx, embedding_matrix)
    return out[:n].reshape(b, s, d)


# speedup vs baseline: 1.8736x; 1.1827x over previous
import functools

import jax
import jax.numpy as jnp
from jax import lax
from jax.experimental import pallas as pl
from jax.experimental.pallas import tpu as pltpu
from jax.experimental.pallas import tpu_sc as plsc

NC = 2
NS = 16
NW = NC * NS
CHUNK = 256
NBUF = 4
LOOKAHEAD = 2


def _gather_kernel(n_rows, d):
    b_per_w = n_rows // NW
    n_chunks = b_per_w // CHUNK
    n_groups = n_chunks // NBUF
    mesh = plsc.VectorSubcoreMesh(core_axis_name="c", subcore_axis_name="s")

    @functools.partial(
        pl.kernel,
        mesh=mesh,
        out_type=jax.ShapeDtypeStruct((n_rows, d), jnp.float32),
        scratch_types=(
            [pltpu.VMEM((b_per_w,), jnp.int32)]
            + [pltpu.VMEM((CHUNK, d), jnp.float32) for _ in range(NBUF)]
            + [pltpu.SemaphoreType.DMA for _ in range(2 * NBUF)]
        ),
        compiler_params=pltpu.CompilerParams(use_tc_tiling_on_sc=False),
    )
    def k(idx_hbm, table_hbm, out_hbm, idx_v, *bufs_and_sems):
        rows = bufs_and_sems[:NBUF]
        gsem = bufs_and_sems[NBUF:2 * NBUF]
        ssem = bufs_and_sems[2 * NBUF:]
        wid = lax.axis_index("s") * NC + lax.axis_index("c")
        base = wid * b_per_w

        def idx_slice(c):
            return idx_v.at[pl.ds(c * CHUNK, CHUNK)]

        def out_slice(c):
            return out_hbm.at[pl.ds(base + c * CHUNK, CHUNK)]

        pltpu.sync_copy(idx_hbm.at[pl.ds(base, b_per_w)], idx_v)

        for j in range(LOOKAHEAD):
            pltpu.async_copy(table_hbm.at[idx_slice(j)], rows[j], gsem[j])

        def body(g, carry):
            for b in range(NBUF):
                i = g * NBUF + b
                j = i + LOOKAHEAD
                jb = (b + LOOKAHEAD) % NBUF

                @pl.when(j < n_chunks)
                def _():
                    @pl.when(j >= NBUF)
                    def _():
                        pltpu.make_async_copy(
                            rows[jb], out_slice(j - NBUF), ssem[jb]
                        ).wait()

                    pltpu.async_copy(
                        table_hbm.at[idx_slice(j)], rows[jb], gsem[jb]
                    )

                pltpu.make_async_copy(
                    table_hbm.at[idx_slice(i)], rows[b], gsem[b]
                ).wait()
                pltpu.async_copy(rows[b], out_slice(i), ssem[b])
            return carry

        lax.fori_loop(0, n_groups, body, 0)

        for b in range(NBUF):
            c = n_chunks - NBUF + b
            pltpu.make_async_copy(rows[b], out_slice(c), ssem[b]).wait()

    return k


def kernel(token_ids, embedding_matrix):
    b, s = token_ids.shape
    d = embedding_matrix.shape[1]
    idx = token_ids.reshape(-1).astype(jnp.int32)
    n = idx.shape[0]
    tile = NW * CHUNK * NBUF
    n_pad = ((n + tile - 1) // tile) * tile
    if n_pad != n:
        idx = jnp.pad(idx, (0, n_pad - n))
    out = _gather_kernel(n_pad, d)(idx, embedding_matrix)
    return out[:n].reshape(b, s, d)
